# baseline (speedup 1.0000x reference)
import functools

import jax
import jax.numpy as jnp
from jax import lax
from jax.experimental import pallas as pl
from jax.experimental.pallas import tpu as pltpu
from jax.experimental.pallas import tpu_sc as plsc

EPS = 1e-5

N = 10000
FD = 128
NP = 10112
NC = 2
NS = 16
NW = NC * NS
K = 128
TPR = NP // NS

@functools.cache
def _mesh():
    return plsc.VectorSubcoreMesh(core_axis_name="c", subcore_axis_name="s",
                                  num_cores=NC, num_subcores=NS)


def _wid():
    return lax.axis_index("s") * NC + lax.axis_index("c")


HB = 16384
NHI = 80


def _prep_body(col2_ref, xp_ref, y_ref, dinvb_ref):
    nch = col2_ref.shape[0]

    def step(i, acc):
        c = col2_ref[i, :]
        hi = c // 128
        lo = c - hi * 128
        oh_hi = (hi[:, None] == lax.broadcasted_iota(jnp.int32, (HB, NHI), 1)
                 ).astype(jnp.bfloat16)
        oh_lo = (lo[:, None] == lax.broadcasted_iota(jnp.int32, (HB, FD), 1)
                 ).astype(jnp.bfloat16)
        return acc + lax.dot_general(
            oh_hi, oh_lo, (((0,), (0,)), ((), ())),
            preferred_element_type=jnp.float32)

    deg2d = lax.fori_loop(
        0, nch, step, jnp.zeros((NHI, FD), jnp.float32))
    dinv2d = jnp.where(deg2d > 0, lax.rsqrt(deg2d), 0.0)
    r_hi = lax.broadcasted_iota(jnp.int32, (NP, NHI), 0) // 128
    oh_rhi = (r_hi == lax.broadcasted_iota(jnp.int32, (NP, NHI), 1)
              ).astype(jnp.float32)
    g = lax.dot_general(oh_rhi, dinv2d, (((1,), (0,)), ((), ())),
                        preferred_element_type=jnp.float32)
    r_lo = (lax.broadcasted_iota(jnp.int32, (NP, FD), 0)
            - 128 * (lax.broadcasted_iota(jnp.int32, (NP, FD), 0) // 128))
    oh_rlo = (r_lo == lax.broadcasted_iota(jnp.int32, (NP, FD), 1)
              ).astype(jnp.float32)
    dinv_col = jnp.sum(g * oh_rlo, axis=1, keepdims=True)
    y_ref[...] = xp_ref[...] * dinv_col
    dinvb_ref[...] = jnp.broadcast_to(dinv_col, (NP, FD))


def _prep_call(col2, xp):
    return pl.pallas_call(
        _prep_body,
        out_shape=(jax.ShapeDtypeStruct((NP, FD), jnp.float32),
                   jax.ShapeDtypeStruct((NP, FD), jnp.float32)),
    )(col2, xp)


BB = 16


def _ax_body(row_hbm, col_hbm, y_hbm, zax_hbm, out_hbm,
             rivA, civA, rivB, civB, rows, sh_ax, semG, semR, semC):
    nb = col_hbm.shape[1] - 1
    c = lax.axis_index("c")
    s = lax.axis_index("s")
    w = _wid()
    pltpu.sync_copy(zax_hbm.at[pl.ds(s * TPR, TPR)],
                    sh_ax.at[pl.ds(s * TPR, TPR)])
    plsc.subcore_barrier()
    pltpu.sync_copy(row_hbm.at[w, 0], rivA)
    pltpu.sync_copy(col_hbm.at[w, 0], civA)

    def body(j, carry):
        b0 = 2 * j
        dBr = pltpu.async_copy(row_hbm.at[w, b0 + 1], rivB, semR)
        dBc = pltpu.async_copy(col_hbm.at[w, b0 + 1], civB, semC)
        pltpu.async_copy(y_hbm.at[rivA], rows, semG).wait()
        pltpu.sync_copy(rows, sh_ax.at[civA], add=True)
        dBr.wait()
        dBc.wait()
        dAr = pltpu.async_copy(row_hbm.at[w, b0 + 2], rivA, semR)
        dAc = pltpu.async_copy(col_hbm.at[w, b0 + 2], civA, semC)
        pltpu.async_copy(y_hbm.at[rivB], rows, semG).wait()
        pltpu.sync_copy(rows, sh_ax.at[civB], add=True)
        dAr.wait()
        dAc.wait()
        return carry

    lax.fori_loop(0, nb // 2, body, 0)
    plsc.subcore_barrier()
    pltpu.sync_copy(sh_ax.at[pl.ds(s * TPR, TPR)],
                    out_hbm.at[c, pl.ds(s * TPR, TPR)])


def _ax_call(row_hbm, col_hbm, y, zax):
    f = functools.partial(
        pl.kernel,
        out_type=jax.ShapeDtypeStruct((NC, NP, FD), jnp.float32),
        mesh=_mesh(),
        scratch_types=[
            pltpu.VMEM((K,), jnp.int32),
            pltpu.VMEM((K,), jnp.int32),
            pltpu.VMEM((K,), jnp.int32),
            pltpu.VMEM((K,), jnp.int32),
            pltpu.VMEM((K, FD), jnp.float32),
            pltpu.VMEM_SHARED((NP, FD), jnp.float32),
            pltpu.SemaphoreType.DMA,
            pltpu.SemaphoreType.DMA,
            pltpu.SemaphoreType.DMA,
        ],
    )(_ax_body)
    return f(row_hbm, col_hbm, y, zax)


def _mlp2(h, W1, b1, g, bt, W2, b2):
    h = jnp.dot(h, W1, preferred_element_type=jnp.float32) + b1
    mean = jnp.mean(h, axis=0, keepdims=True)
    var = jnp.mean((h - mean) ** 2, axis=0, keepdims=True)
    h = (h - mean) * lax.rsqrt(var + EPS) * g + bt
    h = jnp.maximum(h, 0.0)
    return jnp.dot(h, W2, preferred_element_type=jnp.float32) + b2


def _dense_body(x_ref, axp_ref, dinvb_ref,
                Wx1_ref, bx1_ref, gx1_ref, btx1_ref, Wx2_ref, bx2_ref,
                Wa1_ref, ba1_ref, ga1_ref, bta1_ref, Wa2_ref, ba2_ref,
                Wf1_ref, bf1_ref, gf1_ref, btf1_ref, Wf2_ref, bf2_ref,
                out_ref):
    x = x_ref[...]
    ax = (axp_ref[0, :N, :] + axp_ref[1, :N, :]) * dinvb_ref[:N, :]
    h_x = _mlp2(x, Wx1_ref[...], bx1_ref[...], gx1_ref[...], btx1_ref[...],
                Wx2_ref[...], bx2_ref[...])
    h_a = _mlp2(ax, Wa1_ref[...], ba1_ref[...], ga1_ref[...], bta1_ref[...],
                Wa2_ref[...], ba2_ref[...])
    h = jnp.concatenate([h_x, h_a], axis=-1)
    out_ref[...] = _mlp2(h, Wf1_ref[...], bf1_ref[...], gf1_ref[...],
                         btf1_ref[...], Wf2_ref[...], bf2_ref[...])


def _dense_call(x, axp, dinvb, *weights):
    return pl.pallas_call(
        _dense_body,
        out_shape=jax.ShapeDtypeStruct((N, FD), jnp.float32),
    )(x, axp, dinvb, *weights)


def kernel(x, edge_index, Wx1, bx1, gx1, btx1, Wx2, bx2,
           Wa1, ba1, ga1, bta1, Wa2, ba2, Wf1, bf1, gf1, btf1, Wf2, bf2):
    e = edge_index.shape[1]
    nb = 2 * (-(-e // (NW * K * 2)))
    ep = NW * (nb + 1) * K
    row = edge_index[0]
    col = edge_index[1]
    pad = jnp.full((ep - e,), N, jnp.int32)
    def _strips(v):
        return jnp.concatenate([v, pad[:NW * nb * K - e]]).reshape(NW, nb, K)
    row_hbm = jnp.concatenate(
        [_strips(row), jnp.full((NW, 1, K), N, jnp.int32)], axis=1)
    col_hbm = jnp.concatenate(
        [_strips(col), jnp.full((NW, 1, K), N, jnp.int32)], axis=1)
    zax = jnp.zeros((NP, FD), jnp.float32)
    xp = jnp.pad(x, ((0, NP - N), (0, 0)))
    nch = -(-e // HB)
    pad2 = jnp.full((nch * HB - e,), N, jnp.int32)
    col2 = jnp.concatenate([col, pad2]).reshape(nch, HB)

    y, dinvb = _prep_call(col2, xp)
    axp = _ax_call(row_hbm, col_hbm, y, zax)

    r = lambda v: v.reshape(1, -1)
    return _dense_call(
        x, axp, dinvb,
        Wx1, r(bx1), r(gx1), r(btx1), Wx2, r(bx2),
        Wa1, r(ba1), r(ga1), r(bta1), Wa2, r(ba2),
        Wf1, r(bf1), r(gf1), r(btf1), Wf2, r(bf2))


# speedup vs baseline: 1.4326x; 1.4326x over previous
import functools

import jax
import jax.numpy as jnp
from jax import lax
from jax.experimental import pallas as pl
from jax.experimental.pallas import tpu as pltpu
from jax.experimental.pallas import tpu_sc as plsc

EPS = 1e-5

N = 10000
FD = 128
NP = 10112
NC = 2
NS = 16
NW = NC * NS
K = 128
TPR = NP // NS

@functools.cache
def _mesh():
    return plsc.VectorSubcoreMesh(core_axis_name="c", subcore_axis_name="s",
                                  num_cores=NC, num_subcores=NS)


def _wid():
    return lax.axis_index("s") * NC + lax.axis_index("c")


HB = 16384
NHI = 80


def _prep_body(col2_ref, xp_ref, y_ref, dinvb_ref):
    nch = col2_ref.shape[0]

    def step(i, acc):
        c = col2_ref[i, :]
        hi = c // 128
        lo = c - hi * 128
        oh_hi = (hi[:, None] == lax.broadcasted_iota(jnp.int32, (HB, NHI), 1)
                 ).astype(jnp.bfloat16)
        oh_lo = (lo[:, None] == lax.broadcasted_iota(jnp.int32, (HB, FD), 1)
                 ).astype(jnp.bfloat16)
        return acc + lax.dot_general(
            oh_hi, oh_lo, (((0,), (0,)), ((), ())),
            preferred_element_type=jnp.float32)

    deg2d = lax.fori_loop(
        0, nch, step, jnp.zeros((NHI, FD), jnp.float32))
    dinv2d = jnp.where(deg2d > 0, lax.rsqrt(deg2d), 0.0)
    r_hi = lax.broadcasted_iota(jnp.int32, (NP, NHI), 0) // 128
    oh_rhi = (r_hi == lax.broadcasted_iota(jnp.int32, (NP, NHI), 1)
              ).astype(jnp.float32)
    g = lax.dot_general(oh_rhi, dinv2d, (((1,), (0,)), ((), ())),
                        preferred_element_type=jnp.float32)
    r_lo = (lax.broadcasted_iota(jnp.int32, (NP, FD), 0)
            - 128 * (lax.broadcasted_iota(jnp.int32, (NP, FD), 0) // 128))
    oh_rlo = (r_lo == lax.broadcasted_iota(jnp.int32, (NP, FD), 1)
              ).astype(jnp.float32)
    dinv_col = jnp.sum(g * oh_rlo, axis=1, keepdims=True)
    y_ref[...] = xp_ref[...] * dinv_col
    dinvb_ref[...] = jnp.broadcast_to(dinv_col, (NP, FD))


def _prep_call(col2, xp):
    return pl.pallas_call(
        _prep_body,
        out_shape=(jax.ShapeDtypeStruct((NP, FD), jnp.float32),
                   jax.ShapeDtypeStruct((NP, FD), jnp.float32)),
    )(col2, xp)


BB = 16


def _ax_body(row_hbm, col_hbm, y_hbm, zax_hbm, out_hbm,
             riv, civ, rows, sh_ax, sem):
    nb = col_hbm.shape[1]
    c = lax.axis_index("c")
    s = lax.axis_index("s")
    w = _wid()
    pltpu.sync_copy(zax_hbm.at[pl.ds(s * TPR, TPR)],
                    sh_ax.at[pl.ds(s * TPR, TPR)])
    plsc.subcore_barrier()

    def body(b, carry):
        pltpu.sync_copy(row_hbm.at[w, b], riv)
        pltpu.sync_copy(col_hbm.at[w, b], civ)
        pltpu.async_copy(y_hbm.at[riv], rows, sem).wait()
        pltpu.sync_copy(rows, sh_ax.at[civ], add=True)
        return carry

    lax.fori_loop(0, nb, body, 0)
    plsc.subcore_barrier()
    pltpu.sync_copy(sh_ax.at[pl.ds(s * TPR, TPR)],
                    out_hbm.at[c, pl.ds(s * TPR, TPR)])


def _ax_call(row_hbm, col_hbm, y, zax):
    f = functools.partial(
        pl.kernel,
        out_type=jax.ShapeDtypeStruct((NC, NP, FD), jnp.float32),
        mesh=_mesh(),
        scratch_types=[
            pltpu.VMEM((K,), jnp.int32),
            pltpu.VMEM((K,), jnp.int32),
            pltpu.VMEM((K, FD), jnp.float32),
            pltpu.VMEM_SHARED((NP, FD), jnp.float32),
            pltpu.SemaphoreType.DMA,
        ],
    )(_ax_body)
    return f(row_hbm, col_hbm, y, zax)


def _mlp2(h, W1, b1, g, bt, W2, b2):
    h = jnp.dot(h, W1, preferred_element_type=jnp.float32) + b1
    mean = jnp.mean(h, axis=0, keepdims=True)
    var = jnp.mean((h - mean) ** 2, axis=0, keepdims=True)
    h = (h - mean) * lax.rsqrt(var + EPS) * g + bt
    h = jnp.maximum(h, 0.0)
    return jnp.dot(h, W2, preferred_element_type=jnp.float32) + b2


def _dense_body(x_ref, axp_ref, dinvb_ref,
                Wx1_ref, bx1_ref, gx1_ref, btx1_ref, Wx2_ref, bx2_ref,
                Wa1_ref, ba1_ref, ga1_ref, bta1_ref, Wa2_ref, ba2_ref,
                Wf1_ref, bf1_ref, gf1_ref, btf1_ref, Wf2_ref, bf2_ref,
                out_ref):
    x = x_ref[...]
    ax = (axp_ref[0, :N, :] + axp_ref[1, :N, :]) * dinvb_ref[:N, :]
    h_x = _mlp2(x, Wx1_ref[...], bx1_ref[...], gx1_ref[...], btx1_ref[...],
                Wx2_ref[...], bx2_ref[...])
    h_a = _mlp2(ax, Wa1_ref[...], ba1_ref[...], ga1_ref[...], bta1_ref[...],
                Wa2_ref[...], ba2_ref[...])
    h = jnp.concatenate([h_x, h_a], axis=-1)
    out_ref[...] = _mlp2(h, Wf1_ref[...], bf1_ref[...], gf1_ref[...],
                         btf1_ref[...], Wf2_ref[...], bf2_ref[...])


def _dense_call(x, axp, dinvb, *weights):
    return pl.pallas_call(
        _dense_body,
        out_shape=jax.ShapeDtypeStruct((N, FD), jnp.float32),
    )(x, axp, dinvb, *weights)


def kernel(x, edge_index, Wx1, bx1, gx1, btx1, Wx2, bx2,
           Wa1, ba1, ga1, bta1, Wa2, ba2, Wf1, bf1, gf1, btf1, Wf2, bf2):
    e = edge_index.shape[1]
    nb = -(-e // (NW * K))
    ep = NW * nb * K
    row = edge_index[0]
    col = edge_index[1]
    pad = jnp.full((ep - e,), N, jnp.int32)
    row_hbm = jnp.concatenate([row, pad]).reshape(NW, nb, K)
    col_hbm = jnp.concatenate([col, pad]).reshape(NW, nb, K)
    zax = jnp.zeros((NP, FD), jnp.float32)
    xp = jnp.pad(x, ((0, NP - N), (0, 0)))
    nch = -(-e // HB)
    pad2 = jnp.full((nch * HB - e,), N, jnp.int32)
    col2 = jnp.concatenate([col, pad2]).reshape(nch, HB)

    y, dinvb = _prep_call(col2, xp)
    axp = _ax_call(row_hbm, col_hbm, y, zax)

    r = lambda v: v.reshape(1, -1)
    return _dense_call(
        x, axp, dinvb,
        Wx1, r(bx1), r(gx1), r(btx1), Wx2, r(bx2),
        Wa1, r(ba1), r(ga1), r(bta1), Wa2, r(ba2),
        Wf1, r(bf1), r(gf1), r(btf1), Wf2, r(bf2))


# speedup vs baseline: 1.4327x; 1.0001x over previous
import functools

import jax
import jax.numpy as jnp
from jax import lax
from jax.experimental import pallas as pl
from jax.experimental.pallas import tpu as pltpu
from jax.experimental.pallas import tpu_sc as plsc

EPS = 1e-5

N = 10000
FD = 128
NP = 10112
NC = 2
NS = 16
NW = NC * NS
K = 128
TPR = NP // NS

@functools.cache
def _mesh():
    return plsc.VectorSubcoreMesh(core_axis_name="c", subcore_axis_name="s",
                                  num_cores=NC, num_subcores=NS)


def _wid():
    return lax.axis_index("s") * NC + lax.axis_index("c")


HB = 16384
NHI = 80


def _prep_body(col2_ref, xp_ref, y_ref, dinvb_ref):
    nch = col2_ref.shape[0]

    def step(i, acc):
        c = col2_ref[i, :]
        hi = c // 128
        lo = c - hi * 128
        oh_hi = (hi[:, None] == lax.broadcasted_iota(jnp.int32, (HB, NHI), 1)
                 ).astype(jnp.bfloat16)
        oh_lo = (lo[:, None] == lax.broadcasted_iota(jnp.int32, (HB, FD), 1)
                 ).astype(jnp.bfloat16)
        return acc + lax.dot_general(
            oh_hi, oh_lo, (((0,), (0,)), ((), ())),
            preferred_element_type=jnp.float32)

    deg2d = lax.fori_loop(
        0, nch, step, jnp.zeros((NHI, FD), jnp.float32))
    dinv2d = jnp.where(deg2d > 0, lax.rsqrt(deg2d), 0.0)
    r_hi = lax.broadcasted_iota(jnp.int32, (NP, NHI), 0) // 128
    oh_rhi = (r_hi == lax.broadcasted_iota(jnp.int32, (NP, NHI), 1)
              ).astype(jnp.float32)
    g = lax.dot_general(oh_rhi, dinv2d, (((1,), (0,)), ((), ())),
                        preferred_element_type=jnp.float32)
    r_lo = (lax.broadcasted_iota(jnp.int32, (NP, FD), 0)
            - 128 * (lax.broadcasted_iota(jnp.int32, (NP, FD), 0) // 128))
    oh_rlo = (r_lo == lax.broadcasted_iota(jnp.int32, (NP, FD), 1)
              ).astype(jnp.float32)
    dinv_col = jnp.sum(g * oh_rlo, axis=1, keepdims=True)
    y_ref[...] = xp_ref[...] * dinv_col
    dinvb_ref[...] = jnp.broadcast_to(dinv_col, (NP, FD))


def _prep_call(col2, xp):
    return pl.pallas_call(
        _prep_body,
        out_shape=(jax.ShapeDtypeStruct((NP, FD), jnp.float32),
                   jax.ShapeDtypeStruct((NP, FD), jnp.float32)),
    )(col2, xp)


def _ax_body(row_hbm, col_hbm, y_hbm, zax_hbm, out_hbm,
             riv, civ, rows, sh_ax, sem):
    nb = col_hbm.shape[1]
    c = lax.axis_index("c")
    s = lax.axis_index("s")
    w = _wid()
    pltpu.sync_copy(zax_hbm.at[pl.ds(s * TPR, TPR)],
                    sh_ax.at[pl.ds(s * TPR, TPR)])
    plsc.subcore_barrier()

    def body(b, carry):
        pltpu.sync_copy(row_hbm.at[w, b], riv)
        pltpu.sync_copy(col_hbm.at[w, b], civ)
        pltpu.async_copy(y_hbm.at[riv], rows, sem).wait()
        pltpu.sync_copy(rows, sh_ax.at[civ], add=True)
        return carry

    lax.fori_loop(0, nb, body, 0)
    plsc.subcore_barrier()
    pltpu.sync_copy(sh_ax.at[pl.ds(s * TPR, TPR)],
                    out_hbm.at[c, pl.ds(s * TPR, TPR)])


def _ax_call(row_hbm, col_hbm, y, zax):
    f = functools.partial(
        pl.kernel,
        out_type=jax.ShapeDtypeStruct((NC, NP, FD), jnp.float32),
        mesh=_mesh(),
        scratch_types=[
            pltpu.VMEM((K,), jnp.int32),
            pltpu.VMEM((K,), jnp.int32),
            pltpu.VMEM((K, FD), jnp.float32),
            pltpu.VMEM_SHARED((NP, FD), jnp.float32),
            pltpu.SemaphoreType.DMA,
        ],
    )(_ax_body)
    return f(row_hbm, col_hbm, y, zax)


def _mlp2(h, W1, b1, g, bt, W2, b2):
    h = jnp.dot(h, W1, preferred_element_type=jnp.float32) + b1
    mean = jnp.mean(h, axis=0, keepdims=True)
    var = jnp.mean((h - mean) ** 2, axis=0, keepdims=True)
    h = (h - mean) * lax.rsqrt(var + EPS) * g + bt
    h = jnp.maximum(h, 0.0)
    return jnp.dot(h, W2, preferred_element_type=jnp.float32) + b2


def _dense_body(x_ref, axp_ref, dinvb_ref,
                Wx1_ref, bx1_ref, gx1_ref, btx1_ref, Wx2_ref, bx2_ref,
                Wa1_ref, ba1_ref, ga1_ref, bta1_ref, Wa2_ref, ba2_ref,
                Wf1_ref, bf1_ref, gf1_ref, btf1_ref, Wf2_ref, bf2_ref,
                out_ref):
    x = x_ref[...]
    ax = (axp_ref[0, :N, :] + axp_ref[1, :N, :]) * dinvb_ref[:N, :]
    h_x = _mlp2(x, Wx1_ref[...], bx1_ref[...], gx1_ref[...], btx1_ref[...],
                Wx2_ref[...], bx2_ref[...])
    h_a = _mlp2(ax, Wa1_ref[...], ba1_ref[...], ga1_ref[...], bta1_ref[...],
                Wa2_ref[...], ba2_ref[...])
    h = jnp.concatenate([h_x, h_a], axis=-1)
    out_ref[...] = _mlp2(h, Wf1_ref[...], bf1_ref[...], gf1_ref[...],
                         btf1_ref[...], Wf2_ref[...], bf2_ref[...])


def _dense_call(x, axp, dinvb, *weights):
    return pl.pallas_call(
        _dense_body,
        out_shape=jax.ShapeDtypeStruct((N, FD), jnp.float32),
    )(x, axp, dinvb, *weights)


def kernel(x, edge_index, Wx1, bx1, gx1, btx1, Wx2, bx2,
           Wa1, ba1, ga1, bta1, Wa2, ba2, Wf1, bf1, gf1, btf1, Wf2, bf2):
    e = edge_index.shape[1]
    nb = -(-e // (NW * K))
    ep = NW * nb * K
    row = edge_index[0]
    col = edge_index[1]
    pad = jnp.full((ep - e,), N, jnp.int32)
    row_hbm = jnp.concatenate([row, pad]).reshape(NW, nb, K)
    col_hbm = jnp.concatenate([col, pad]).reshape(NW, nb, K)
    zax = jnp.zeros((NP, FD), jnp.float32)
    xp = jnp.pad(x, ((0, NP - N), (0, 0)))
    nch = -(-e // HB)
    pad2 = jnp.full((nch * HB - e,), N, jnp.int32)
    col2 = jnp.concatenate([col, pad2]).reshape(nch, HB)

    y, dinvb = _prep_call(col2, xp)
    axp = _ax_call(row_hbm, col_hbm, y, zax)

    r = lambda v: v.reshape(1, -1)
    return _dense_call(
        x, axp, dinvb,
        Wx1, r(bx1), r(gx1), r(btx1), Wx2, r(bx2),
        Wa1, r(ba1), r(ga1), r(bta1), Wa2, r(ba2),
        Wf1, r(bf1), r(gf1), r(btf1), Wf2, r(bf2))


# speedup vs baseline: 1.6090x; 1.1231x over previous
import functools

import jax
import jax.numpy as jnp
from jax import lax
from jax.experimental import pallas as pl
from jax.experimental.pallas import tpu as pltpu
from jax.experimental.pallas import tpu_sc as plsc

EPS = 1e-5

N = 10000
FD = 128
NP = 10112
NC = 2
NS = 16
NW = NC * NS
K = 128
TPR = NP // NS

@functools.cache
def _mesh():
    return plsc.VectorSubcoreMesh(core_axis_name="c", subcore_axis_name="s",
                                  num_cores=NC, num_subcores=NS)


def _wid():
    return lax.axis_index("s") * NC + lax.axis_index("c")


HB = 16384
NHI = 80


def _prep_body(col2_ref, xp_ref, y_ref, dinvb_ref):
    nch = col2_ref.shape[0]

    def step(i, acc):
        c = col2_ref[i, :]
        hi = c // 128
        lo = c - hi * 128
        oh_hi = (hi[:, None] == lax.broadcasted_iota(jnp.int32, (HB, NHI), 1)
                 ).astype(jnp.bfloat16)
        oh_lo = (lo[:, None] == lax.broadcasted_iota(jnp.int32, (HB, FD), 1)
                 ).astype(jnp.bfloat16)
        return acc + lax.dot_general(
            oh_hi, oh_lo, (((0,), (0,)), ((), ())),
            preferred_element_type=jnp.float32)

    deg2d = lax.fori_loop(
        0, nch, step, jnp.zeros((NHI, FD), jnp.float32))
    dinv2d = jnp.where(deg2d > 0, lax.rsqrt(deg2d), 0.0)
    r_hi = lax.broadcasted_iota(jnp.int32, (NP, NHI), 0) // 128
    oh_rhi = (r_hi == lax.broadcasted_iota(jnp.int32, (NP, NHI), 1)
              ).astype(jnp.float32)
    g = lax.dot_general(oh_rhi, dinv2d, (((1,), (0,)), ((), ())),
                        preferred_element_type=jnp.float32)
    r_lo = (lax.broadcasted_iota(jnp.int32, (NP, FD), 0)
            - 128 * (lax.broadcasted_iota(jnp.int32, (NP, FD), 0) // 128))
    oh_rlo = (r_lo == lax.broadcasted_iota(jnp.int32, (NP, FD), 1)
              ).astype(jnp.float32)
    dinv_col = jnp.sum(g * oh_rlo, axis=1, keepdims=True)
    y_ref[...] = xp_ref[...] * dinv_col
    dinvb_ref[...] = jnp.broadcast_to(dinv_col, (NP, FD))


def _prep_call(col2, xp):
    return pl.pallas_call(
        _prep_body,
        out_shape=(jax.ShapeDtypeStruct((NP, FD), jnp.float32),
                   jax.ShapeDtypeStruct((NP, FD), jnp.float32)),
    )(col2, xp)


def _ax_body(n0, n1, row_hbm, col_hbm, y_hbm, zax_hbm, out_hbm,
             riv, civ, rows, sh_ax, sem):
    c = lax.axis_index("c")
    s = lax.axis_index("s")
    w = _wid()
    nb = jnp.where(c == 0, n0, n1)
    pltpu.sync_copy(zax_hbm.at[pl.ds(s * TPR, TPR)],
                    sh_ax.at[pl.ds(s * TPR, TPR)])
    plsc.subcore_barrier()

    def body(b, carry):
        pltpu.sync_copy(row_hbm.at[w, b], riv)
        pltpu.sync_copy(col_hbm.at[w, b], civ)
        pltpu.async_copy(y_hbm.at[riv], rows, sem).wait()
        pltpu.sync_copy(rows, sh_ax.at[civ], add=True)
        return carry

    lax.fori_loop(0, nb, body, 0)
    plsc.subcore_barrier()
    pltpu.sync_copy(sh_ax.at[pl.ds(s * TPR, TPR)],
                    out_hbm.at[c, pl.ds(s * TPR, TPR)])


def _ax_call(n0, n1, row_hbm, col_hbm, y, zax):
    f = functools.partial(
        pl.kernel,
        out_type=jax.ShapeDtypeStruct((NC, NP, FD), jnp.float32),
        mesh=_mesh(),
        scratch_types=[
            pltpu.VMEM((K,), jnp.int32),
            pltpu.VMEM((K,), jnp.int32),
            pltpu.VMEM((K, FD), jnp.float32),
            pltpu.VMEM_SHARED((NP, FD), jnp.float32),
            pltpu.SemaphoreType.DMA,
        ],
    )(functools.partial(_ax_body, n0, n1))
    return f(row_hbm, col_hbm, y, zax)


def _mlp2(h, W1, b1, g, bt, W2, b2):
    h = jnp.dot(h, W1, preferred_element_type=jnp.float32) + b1
    mean = jnp.mean(h, axis=0, keepdims=True)
    var = jnp.mean((h - mean) ** 2, axis=0, keepdims=True)
    h = (h - mean) * lax.rsqrt(var + EPS) * g + bt
    h = jnp.maximum(h, 0.0)
    return jnp.dot(h, W2, preferred_element_type=jnp.float32) + b2


def _dense_body(x_ref, axp_ref, dinvb_ref,
                Wx1_ref, bx1_ref, gx1_ref, btx1_ref, Wx2_ref, bx2_ref,
                Wa1_ref, ba1_ref, ga1_ref, bta1_ref, Wa2_ref, ba2_ref,
                Wf1_ref, bf1_ref, gf1_ref, btf1_ref, Wf2_ref, bf2_ref,
                out_ref):
    x = x_ref[...]
    ax = (axp_ref[0, :N, :] + axp_ref[1, :N, :]) * dinvb_ref[:N, :]
    h_x = _mlp2(x, Wx1_ref[...], bx1_ref[...], gx1_ref[...], btx1_ref[...],
                Wx2_ref[...], bx2_ref[...])
    h_a = _mlp2(ax, Wa1_ref[...], ba1_ref[...], ga1_ref[...], bta1_ref[...],
                Wa2_ref[...], ba2_ref[...])
    h = jnp.concatenate([h_x, h_a], axis=-1)
    out_ref[...] = _mlp2(h, Wf1_ref[...], bf1_ref[...], gf1_ref[...],
                         btf1_ref[...], Wf2_ref[...], bf2_ref[...])


def _dense_call(x, axp, dinvb, *weights):
    return pl.pallas_call(
        _dense_body,
        out_shape=jax.ShapeDtypeStruct((N, FD), jnp.float32),
    )(x, axp, dinvb, *weights)


def kernel(x, edge_index, Wx1, bx1, gx1, btx1, Wx2, bx2,
           Wa1, ba1, ga1, bta1, Wa2, ba2, Wf1, bf1, gf1, btf1, Wf2, bf2):
    e = edge_index.shape[1]
    ntot = -(-e // (NS * K))
    n0 = (ntot * 94 + 79) // 158
    n1 = ntot - n0
    ep = NS * ntot * K
    row = edge_index[0]
    col = edge_index[1]
    pad = jnp.full((ep - e,), N, jnp.int32)

    def _strips(v):
        vp = jnp.concatenate([v, pad])
        b0 = vp[:NS * n0 * K].reshape(NS, 1, n0, K)
        b1 = jnp.concatenate(
            [vp[NS * n0 * K:].reshape(NS, 1, n1, K),
             jnp.full((NS, 1, n0 - n1, K), N, jnp.int32)], axis=2)
        return jnp.concatenate([b0, b1], axis=1).reshape(NW, n0, K)

    row_hbm = _strips(row)
    col_hbm = _strips(col)
    zax = jnp.zeros((NP, FD), jnp.float32)
    xp = jnp.pad(x, ((0, NP - N), (0, 0)))
    nch = -(-e // HB)
    pad2 = jnp.full((nch * HB - e,), N, jnp.int32)
    col2 = jnp.concatenate([col, pad2]).reshape(nch, HB)

    y, dinvb = _prep_call(col2, xp)
    axp = _ax_call(n0, n1, row_hbm, col_hbm, y, zax)

    r = lambda v: v.reshape(1, -1)
    return _dense_call(
        x, axp, dinvb,
        Wx1, r(bx1), r(gx1), r(btx1), Wx2, r(bx2),
        Wa1, r(ba1), r(ga1), r(bta1), Wa2, r(ba2),
        Wf1, r(bf1), r(gf1), r(btf1), Wf2, r(bf2))
